# Initial kernel scaffold; baseline (speedup 1.0000x reference)
import jax
import jax.numpy as jnp
from jax.experimental import pallas as pl


def kernel(feat, inc_src, inc_dst, edge_weight, Ws, bs, Wc, bc, Wh1, bh1, Wh2, bh2, Wf, Wfa1, bfa1, Wfa2, bfa2, Wfa3, bfa3):
    raise NotImplementedError("write your pallas kernel here")


# speedup vs baseline: 4.0620x; 4.0620x over previous
import functools

import jax
import jax.numpy as jnp
from jax import lax
from jax.experimental import pallas as pl
from jax.experimental.pallas import tpu as pltpu
from jax.experimental.pallas import tpu_sc as plsc

NPROT = 10000
NHE = 2000
EDGES = 320000
D = 128
LANES = 16
NCORES = 2
NSUB = 16
NWORK = NCORES * NSUB
CHUNK = 128


def _scatter_pass_body(nseg, nchunk):
  rpt = nseg // NSUB

  def body(table_hbm, gidx_hbm, sidx_hbm, w_hbm, out_hbm,
           gidx_v, sidx_v, w_v, rows_v, acc_sh, sem):
    cid = lax.axis_index("c")
    sid = lax.axis_index("s")
    wid = sid * NCORES + cid

    pltpu.sync_copy(gidx_hbm.at[wid], gidx_v)
    pltpu.sync_copy(sidx_hbm.at[wid], sidx_v)
    pltpu.sync_copy(w_hbm.at[wid], w_v)

    zero = jnp.zeros((LANES,), jnp.float32)

    def zrow(i, carry):
      for j in range(D // LANES):
        rows_v[i, pl.ds(j * LANES, LANES)] = zero
      return carry

    lax.fori_loop(0, CHUNK, zrow, 0)

    base = sid * rpt
    off = 0
    while off < rpt:
      n = min(CHUNK, rpt - off)
      pltpu.sync_copy(rows_v.at[pl.ds(0, n)], acc_sh.at[pl.ds(base + off, n)])
      off += n
    plsc.subcore_barrier()

    def chunk_body(ch, carry):
      pltpu.async_copy(table_hbm.at[gidx_v.at[ch]], rows_v, sem).wait()

      def grp_body(g, c2):
        wvec = w_v[ch, pl.ds(g * LANES, LANES)]
        for i in range(LANES):
          w = wvec[i]
          r = g * LANES + i
          for j in range(D // LANES):
            sl = pl.ds(j * LANES, LANES)
            rows_v[r, sl] = rows_v[r, sl] * w
        return c2

      lax.fori_loop(0, CHUNK // LANES, grp_body, 0)
      pltpu.sync_copy(rows_v, acc_sh.at[sidx_v.at[ch]], add=True)
      return carry

    lax.fori_loop(0, nchunk, chunk_body, 0)
    plsc.subcore_barrier()

    off = 0
    while off < rpt:
      n = min(CHUNK, rpt - off)
      pltpu.sync_copy(acc_sh.at[pl.ds(base + off, n)],
                      out_hbm.at[cid, pl.ds(base + off, n)])
      off += n

  return body


@functools.partial(jax.jit, static_argnames=("nseg",))
def _run_scatter_pass(table, gidx, sidx, w, nseg):
  nseg = -(-nseg // (NSUB * 8)) * (NSUB * 8)
  epw = EDGES // NWORK
  nchunk = -(-epw // CHUNK)
  epad = NWORK * nchunk * CHUNK
  pad = epad - EDGES
  gidx_p = jnp.concatenate([gidx, jnp.zeros((pad,), jnp.int32)]).reshape(
      NWORK, nchunk, CHUNK)
  sidx_p = jnp.concatenate([sidx, jnp.zeros((pad,), jnp.int32)]).reshape(
      NWORK, nchunk, CHUNK)
  w_p = jnp.concatenate([w, jnp.zeros((pad,), jnp.float32)]).reshape(
      NWORK, nchunk, CHUNK)

  kfn = pl.kernel(
      _scatter_pass_body(nseg, nchunk),
      out_type=jax.ShapeDtypeStruct((NCORES, nseg, D), jnp.float32),
      mesh=plsc.VectorSubcoreMesh(core_axis_name="c", subcore_axis_name="s",
                                  num_cores=NCORES, num_subcores=NSUB),
      scratch_types=[
          pltpu.VMEM((nchunk, CHUNK), jnp.int32),
          pltpu.VMEM((nchunk, CHUNK), jnp.int32),
          pltpu.VMEM((nchunk, CHUNK), jnp.float32),
          pltpu.VMEM((CHUNK, D), jnp.float32),
          pltpu.VMEM_SHARED((nseg, D), jnp.float32),
          pltpu.SemaphoreType.DMA,
      ],
  )
  return kfn(table, gidx_p, sidx_p, w_p)


def _attn_body(hp_ref, w1_ref, b1_ref, w2_ref, b2_ref, wf_ref, out_ref):
  h = hp_ref[0] + hp_ref[1]
  hh = jnp.maximum(
      jnp.dot(h, w1_ref[...], preferred_element_type=jnp.float32) + b1_ref[...],
      0.0)
  a = jnp.dot(hh, w2_ref[...], preferred_element_type=jnp.float32) + b2_ref[...]
  a = 1.0 / (1.0 + jnp.exp(-a))
  s = jnp.dot(a, wf_ref[...], preferred_element_type=jnp.float32)
  out_ref[...] = h * s


def _final_body(feat_ref, cp_ref, ws_ref, bs_ref, wc_ref, bc_ref,
                w1a_ref, w1b_ref, b1_ref, w2_ref, b2_ref, w3_ref, b3_ref,
                out_ref):
  f = feat_ref[...]
  c = cp_ref[0] + cp_ref[1]
  sf = jnp.dot(f, ws_ref[...], preferred_element_type=jnp.float32) + bs_ref[...]
  ct = jnp.dot(c, wc_ref[...], preferred_element_type=jnp.float32) + bc_ref[...]
  h1 = jnp.maximum(
      jnp.dot(sf, w1a_ref[...], preferred_element_type=jnp.float32)
      + jnp.dot(ct, w1b_ref[...], preferred_element_type=jnp.float32)
      + b1_ref[...], 0.0)
  h2 = jnp.maximum(
      jnp.dot(h1, w2_ref[...], preferred_element_type=jnp.float32)
      + b2_ref[...], 0.0)
  t = jnp.dot(h2, w3_ref[...], preferred_element_type=jnp.float32) + b3_ref[...]
  w0 = 1.0 / (1.0 + jnp.exp(-t))
  fused = sf * w0 + ct * (1.0 - w0) + f
  out_ref[...] = jnp.maximum(fused, 0.0)


def kernel(feat, inc_src, inc_dst, edge_weight,
           Ws, bs, Wc, bc, Wh1, bh1, Wh2, bh2, Wf,
           Wfa1, bfa1, Wfa2, bfa2, Wfa3, bfa3):
  n_heads, head_dim, _ = Wh1.shape

  hp = _run_scatter_pass(feat, inc_src, inc_dst, edge_weight, NHE)[:, :NHE]

  w1 = Wh1.reshape(n_heads * head_dim, D).T
  b1 = bh1.reshape(1, n_heads * head_dim)
  w2 = jax.scipy.linalg.block_diag(*[Wh2[i].T for i in range(n_heads)])
  b2 = bh2.reshape(1, n_heads)
  wf = Wf.T

  hew = pl.pallas_call(
      _attn_body,
      out_shape=jax.ShapeDtypeStruct((NHE, D), jnp.float32),
  )(hp, w1, b1, w2, b2, wf)

  cp = _run_scatter_pass(hew, inc_dst, inc_src, edge_weight, NPROT)[:, :NPROT]

  nblk = max(1, NPROT // 2000)
  blk = NPROT // nblk
  ws_t = Ws.T
  wc_t = Wc.T
  w1a = Wfa1[:, :D].T
  w1b = Wfa1[:, D:].T
  b1f = bfa1.reshape(1, -1)
  w2f = Wfa2.T
  b2f = bfa2.reshape(1, -1)
  w3f = (Wfa3[0] - Wfa3[1]).reshape(-1, 1)
  b3f = (bfa3[0] - bfa3[1]).reshape(1, 1)

  full = lambda shape: pl.BlockSpec(shape, lambda i: (0,) * len(shape))
  out = pl.pallas_call(
      _final_body,
      grid=(nblk,),
      in_specs=[
          pl.BlockSpec((blk, D), lambda i: (i, 0)),
          pl.BlockSpec((NCORES, blk, D), lambda i: (0, i, 0)),
          full((D, D)), full((1, D)), full((D, D)), full((1, D)),
          full((D, D)), full((D, D)), full((1, D)),
          full((D, 64)), full((1, 64)), full((64, 1)), full((1, 1)),
      ],
      out_specs=pl.BlockSpec((blk, D), lambda i: (i, 0)),
      out_shape=jax.ShapeDtypeStruct((NPROT, D), jnp.float32),
  )(feat, cp, ws_t, bs.reshape(1, -1), wc_t, bc.reshape(1, -1),
    w1a, w1b, b1f, w2f, b2f, w3f, b3f)
  return out
